# Initial kernel scaffold; baseline (speedup 1.0000x reference)
import jax
import jax.numpy as jnp
from jax.experimental import pallas as pl


def kernel(inputs, inputs_positions, position_emb):
    raise NotImplementedError("write your pallas kernel here")


# speedup vs baseline: 1.5323x; 1.5323x over previous
import functools

import jax
import jax.numpy as jnp
from jax import lax
from jax.experimental import pallas as pl
from jax.experimental.pallas import tpu as pltpu
from jax.experimental.pallas import tpu_sc as plsc

_NC = 2
_NS = 16
_NW = _NC * _NS
_L = 16
_C = 128


@functools.partial(jax.jit, static_argnames=())
def _sc_gather_add(x, idx, table):
    T, D = x.shape
    R = T // _NW
    chunks = R // _C

    mesh = plsc.VectorSubcoreMesh(core_axis_name="c", subcore_axis_name="s")

    @functools.partial(
        pl.kernel,
        out_type=jax.ShapeDtypeStruct((T, D), jnp.float32),
        mesh=mesh,
        scratch_types=[
            pltpu.VMEM((_C,), jnp.int32),
            pltpu.VMEM((_C, D), jnp.float32),
            pltpu.VMEM((_C, D), jnp.float32),
            pltpu.SemaphoreType.DMA,
        ],
    )
    def body(x_hbm, idx_hbm, tab_hbm, out_hbm, idx_v, in_v, row_v, sem):
        wid = lax.axis_index("s") * _NC + lax.axis_index("c")

        def chunk(i, carry):
            base = wid * R + i * _C
            pltpu.sync_copy(idx_hbm.at[pl.ds(base, _C)], idx_v)
            pltpu.sync_copy(x_hbm.at[pl.ds(base, _C)], in_v)
            pltpu.async_copy(tab_hbm.at[idx_v], row_v, sem).wait()

            def addrow(r, c2):
                for d in range(D // _L):
                    sl = pl.ds(d * _L, _L)
                    in_v[r, sl] = in_v[r, sl] + row_v[r, sl]
                return c2

            lax.fori_loop(0, _C, addrow, 0, unroll=False)
            pltpu.sync_copy(in_v, out_hbm.at[pl.ds(base, _C)])
            return carry

        lax.fori_loop(0, chunks, chunk, 0, unroll=False)

    return body(x, idx, table)


def kernel(inputs, inputs_positions, position_emb):
    B, N, D = inputs.shape
    T = B * N
    x = inputs.reshape(T, D)
    idx = inputs_positions.reshape(T).astype(jnp.int32)
    table = position_emb.reshape(position_emb.shape[1], D)
    out = _sc_gather_add(x, idx, table)
    return out.reshape(B, N, D)


# speedup vs baseline: 1.6064x; 1.0483x over previous
import functools

import jax
import jax.numpy as jnp
from jax import lax
from jax.experimental import pallas as pl
from jax.experimental.pallas import tpu as pltpu
from jax.experimental.pallas import tpu_sc as plsc

_NC = 2
_NS = 16
_NW = _NC * _NS
_L = 16
_C = 32
_NBUF = 4


def _sc_gather_add(x, idx, table):
    T, D = x.shape
    R = T // _NW
    chunks = R // _C
    nk = chunks // _NBUF

    mesh = plsc.VectorSubcoreMesh(core_axis_name="c", subcore_axis_name="s")

    buf = lambda: pltpu.VMEM((_C, D), jnp.float32)

    @functools.partial(
        pl.kernel,
        out_type=jax.ShapeDtypeStruct((T, D), jnp.float32),
        mesh=mesh,
        scratch_types=[
            pltpu.VMEM((R,), jnp.int32),
            buf(), buf(), buf(), buf(),
            buf(), buf(), buf(), buf(),
            pltpu.SemaphoreType.DMA((_NBUF,)),
            pltpu.SemaphoreType.DMA((_NBUF,)),
            pltpu.SemaphoreType.DMA((_NBUF,)),
        ],
    )
    def body(x_hbm, idx_hbm, tab_hbm, out_hbm, idx_all,
             in0, in1, in2, in3, row0, row1, row2, row3,
             sem_in, sem_row, sem_out):
        ins = (in0, in1, in2, in3)
        rows = (row0, row1, row2, row3)
        wid = lax.axis_index("s") * _NC + lax.axis_index("c")
        rbase = wid * R

        pltpu.sync_copy(idx_hbm.at[pl.ds(rbase, R)], idx_all)

        def issue(i, b):
            base = rbase + i * _C
            pltpu.async_copy(x_hbm.at[pl.ds(base, _C)], ins[b], sem_in.at[b])
            pltpu.async_copy(tab_hbm.at[idx_all.at[pl.ds(i * _C, _C)]],
                             rows[b], sem_row.at[b])

        for b in range(_NBUF):
            issue(b, b)

        def k_body(k, carry):
            for b in range(_NBUF):
                i = k * _NBUF + b
                base = rbase + i * _C
                pltpu.make_async_copy(
                    x_hbm.at[pl.ds(base, _C)], ins[b], sem_in.at[b]).wait()
                pltpu.make_async_copy(
                    tab_hbm.at[idx_all.at[pl.ds(i * _C, _C)]],
                    rows[b], sem_row.at[b]).wait()

                def addrow(r, c2, _b=b):
                    for d in range(D // _L):
                        sl = pl.ds(d * _L, _L)
                        ins[_b][r, sl] = ins[_b][r, sl] + rows[_b][r, sl]
                    return c2

                lax.fori_loop(0, _C, addrow, 0, unroll=False)
                pltpu.async_copy(ins[b], out_hbm.at[pl.ds(base, _C)],
                                 sem_out.at[b])

            for b in range(_NBUF):
                i = (k + 1) * _NBUF + b

                @pl.when(i < chunks)
                def _(b=b, i=i):
                    prev = rbase + (i - _NBUF) * _C
                    pltpu.make_async_copy(
                        ins[b], out_hbm.at[pl.ds(prev, _C)],
                        sem_out.at[b]).wait()
                    issue(i, b)

            return carry

        lax.fori_loop(0, nk, k_body, 0, unroll=False)

        for b in range(_NBUF):
            i = (nk - 1) * _NBUF + b
            base = rbase + i * _C
            pltpu.make_async_copy(
                ins[b], out_hbm.at[pl.ds(base, _C)], sem_out.at[b]).wait()

    return body(x, idx, table)


def kernel(inputs, inputs_positions, position_emb):
    B, N, D = inputs.shape
    T = B * N
    x = inputs.reshape(T, D)
    idx = inputs_positions.reshape(T).astype(jnp.int32)
    table = position_emb.reshape(position_emb.shape[1], D)
    out = _sc_gather_add(x, idx, table)
    return out.reshape(B, N, D)
